# baseline (speedup 1.0000x reference)
import jax
import jax.numpy as jnp
from jax.experimental import pallas as pl
from jax.experimental.pallas import tpu as pltpu
from jax.experimental.pallas import tpu_sc as plsc

GS = 32
G = 96
HID = 128
C = 384
N = 16384
SUB = 8
LAN = N // SUB
NB = 2048
BIGF = 3.4e38
DPAD = 128
NC_SC = 2
NS_SC = 16
NW = NC_SC * NS_SC
BPW = GS * G // NW


def _fps_knn_kernel(xyz_ref, x8_ref, y8_ref, z8_ref, center_ref, ori_ref,
                    cidx_ref):
    x8 = x8_ref[:]
    y8 = y8_ref[:]
    z8 = z8_ref[:]
    n_iota8 = (jax.lax.broadcasted_iota(jnp.int32, (SUB, LAN), 0) * LAN
               + jax.lax.broadcasted_iota(jnp.int32, (SUB, LAN), 1))
    s_iota_col = jax.lax.broadcasted_iota(jnp.int32, (G, 1), 0)
    row_iota_g = jax.lax.broadcasted_iota(jnp.int32, (1, G), 1)

    def body(i, carry):
        dists, far, cxs, cys, czs, cidx = carry
        cidx = jnp.where(row_iota_g == i, far, cidx)
        sel = n_iota8 == far
        cx = jnp.sum(jnp.where(sel, x8, 0.0))
        cy = jnp.sum(jnp.where(sel, y8, 0.0))
        cz = jnp.sum(jnp.where(sel, z8, 0.0))
        cxs = jnp.where(s_iota_col == i, cx, cxs)
        cys = jnp.where(s_iota_col == i, cy, cys)
        czs = jnp.where(s_iota_col == i, cz, czs)
        d = (x8 - cx) ** 2 + (y8 - cy) ** 2 + (z8 - cz) ** 2
        dists = jnp.minimum(dists, d)
        far = jnp.argmax(dists).astype(jnp.int32)
        return dists, far, cxs, cys, czs, cidx

    init = (jnp.full((SUB, LAN), 1e10, jnp.float32), jnp.int32(0),
            jnp.zeros((G, 1), jnp.float32), jnp.zeros((G, 1), jnp.float32),
            jnp.zeros((G, 1), jnp.float32), jnp.zeros((1, G), jnp.int32))
    _, _, cxs, cys, czs, cidx = jax.lax.fori_loop(0, G, body, init)

    px = xyz_ref[0:1, :]
    py = xyz_ref[1:2, :]
    pz = xyz_ref[2:3, :]
    d2 = (cxs - px) ** 2 + (cys - py) ** 2 + (czs - pz) ** 2

    lane_iota_n = jax.lax.broadcasted_iota(jnp.int32, (G, N), 1)
    l_iota_gs = jax.lax.broadcasted_iota(jnp.int32, (G, GS), 1)
    ori = jnp.zeros((G, GS), jnp.int32)
    for k in range(GS):
        idxc = jnp.expand_dims(jnp.argmin(d2, axis=1).astype(jnp.int32), 1)
        ori = jnp.where(l_iota_gs == k, idxc, ori)
        d2 = jnp.where(lane_iota_n == idxc, BIGF, d2)

    center_ref[:, 0:1] = cxs
    center_ref[:, 1:2] = cys
    center_ref[:, 2:3] = czs
    ori_ref[:] = ori
    cidx_ref[:] = cidx


def _sc_gather_kernel(table_hbm, idx_hbm, out_hbm, idx_v, rows_v, sem):
    wid = jax.lax.axis_index("s") * NC_SC + jax.lax.axis_index("c")
    base = wid * BPW
    pltpu.sync_copy(idx_hbm.at[pl.ds(base, BPW)], idx_v)
    pltpu.async_copy(table_hbm.at[idx_v], rows_v, sem).wait()
    pltpu.sync_copy(rows_v, out_hbm.at[pl.ds(base, BPW)])


def _mlp_kernel(nb_ref, ctr_ref, w1_ref, b1_ref, w2_ref, b2_ref, fgc_ref):
    nb = nb_ref[:]
    cxs = ctr_ref[:, 0:1]
    cys = ctr_ref[:, 1:2]
    czs = ctr_ref[:, 2:3]
    cxa = jnp.concatenate([cxs] * GS, axis=0)
    cya = jnp.concatenate([cys] * GS, axis=0)
    cza = jnp.concatenate([czs] * GS, axis=0)
    nx = nb[:, 0:1] - cxa
    ny = nb[:, 1:2] - cya
    nz = nb[:, 2:3] - cza
    w1 = w1_ref[:]
    h = jnp.maximum(
        nx * w1[0:1, :] + ny * w1[1:2, :] + nz * w1[2:3, :] + b1_ref[:], 0.0)
    f = jnp.dot(h, w2_ref[:], preferred_element_type=jnp.float32,
                precision=jax.lax.Precision.HIGHEST) + b2_ref[:]
    fgc = f[0:G, :]
    for k in range(1, GS):
        fgc = jnp.maximum(fgc, f[k * G:(k + 1) * G, :])
    fgc_ref[:] = fgc


def _interp_kernel(xyz_ref, fcg_ref, ctr_ref, out_ref):
    px = xyz_ref[0:1, :]
    py = xyz_ref[1:2, :]
    pz = xyz_ref[2:3, :]
    cxs = ctr_ref[:, 0:1]
    cys = ctr_ref[:, 1:2]
    czs = ctr_ref[:, 2:3]
    d2 = (px - cxs) ** 2 + (py - cys) ** 2 + (pz - czs) ** 2
    g_iota = jax.lax.broadcasted_iota(jnp.int32, (G, NB), 0)
    rs = []
    gsel = []
    for _ in range(3):
        m = jnp.min(d2, axis=0, keepdims=True)
        gk = jnp.min(jnp.where(d2 == m, g_iota, G), axis=0, keepdims=True)
        dist = jnp.maximum(m, 1e-10)
        rs.append(1.0 / (dist + 1e-8))
        gsel.append(gk)
        d2 = jnp.where(g_iota == gk, BIGF, d2)
    rsum = (rs[0] + rs[1]) + rs[2]
    w = jnp.zeros((G, NB), jnp.float32)
    for k in range(3):
        w = w + jnp.where(g_iota == gsel[k], rs[k] / rsum, 0.0)
    out_ref[:] = jnp.dot(fcg_ref[:], w, preferred_element_type=jnp.float32,
                         precision=jax.lax.Precision.HIGHEST)


def kernel(xyz, W1, b1, W2, b2):
    x = xyz[0]
    x8 = x[0].reshape(SUB, LAN)
    y8 = x[1].reshape(SUB, LAN)
    z8 = x[2].reshape(SUB, LAN)
    center, ori, cidx = pl.pallas_call(
        _fps_knn_kernel,
        out_shape=[
            jax.ShapeDtypeStruct((G, 3), jnp.float32),
            jax.ShapeDtypeStruct((G, GS), jnp.int32),
            jax.ShapeDtypeStruct((1, G), jnp.int32),
        ],
    )(x, x8, y8, z8)

    table = jnp.concatenate(
        [x.T, jnp.zeros((N, DPAD - 3), jnp.float32)], axis=1)
    idx_flat = ori.T.reshape(GS * G)
    nb = pl.kernel(
        _sc_gather_kernel,
        out_type=jax.ShapeDtypeStruct((GS * G, DPAD), jnp.float32),
        mesh=plsc.VectorSubcoreMesh(core_axis_name="c", subcore_axis_name="s"),
        scratch_types=[
            pltpu.VMEM((BPW,), jnp.int32),
            pltpu.VMEM((BPW, DPAD), jnp.float32),
            pltpu.SemaphoreType.DMA,
        ],
    )(table, idx_flat)

    fgc = pl.pallas_call(
        _mlp_kernel,
        out_shape=jax.ShapeDtypeStruct((G, C), jnp.float32),
    )(nb, center, W1, b1.reshape(1, HID), W2, b2.reshape(1, C))

    fcg = fgc.T
    interp = pl.pallas_call(
        _interp_kernel,
        grid=(N // NB,),
        in_specs=[
            pl.BlockSpec((3, NB), lambda i: (0, i)),
            pl.BlockSpec((C, G), lambda i: (0, 0)),
            pl.BlockSpec((G, 3), lambda i: (0, 0)),
        ],
        out_specs=pl.BlockSpec((C, NB), lambda i: (0, i)),
        out_shape=jax.ShapeDtypeStruct((C, N), jnp.float32),
    )(x, fcg, center)
    return (fcg[None], center[None], ori[None], cidx, interp[None])


# speedup vs baseline: 1.1231x; 1.1231x over previous
import jax
import jax.numpy as jnp
from jax.experimental import pallas as pl
from jax.experimental.pallas import tpu as pltpu
from jax.experimental.pallas import tpu_sc as plsc

GS = 32
G = 96
HID = 128
C = 384
N = 16384
SUB = 8
LAN = N // SUB
NB = 2048
BIGF = 3.4e38
DPAD = 128
NC_SC = 2
NS_SC = 16
NW = NC_SC * NS_SC
BPW = GS * G // NW


def _fps_knn_kernel(xyz_ref, x8_ref, y8_ref, z8_ref, center_ref, ori_ref,
                    cidx_ref):
    x8 = x8_ref[:]
    y8 = y8_ref[:]
    z8 = z8_ref[:]
    n_iota8 = (jax.lax.broadcasted_iota(jnp.int32, (SUB, LAN), 0) * LAN
               + jax.lax.broadcasted_iota(jnp.int32, (SUB, LAN), 1))
    s_iota_col = jax.lax.broadcasted_iota(jnp.int32, (G, 1), 0)
    row_iota_g = jax.lax.broadcasted_iota(jnp.int32, (1, G), 1)

    def body(i, carry):
        dists, far, cxs, cys, czs, cidx = carry
        cidx = jnp.where(row_iota_g == i, far, cidx)
        sel = n_iota8 == far
        cx = jnp.sum(jnp.where(sel, x8, 0.0))
        cy = jnp.sum(jnp.where(sel, y8, 0.0))
        cz = jnp.sum(jnp.where(sel, z8, 0.0))
        cxs = jnp.where(s_iota_col == i, cx, cxs)
        cys = jnp.where(s_iota_col == i, cy, cys)
        czs = jnp.where(s_iota_col == i, cz, czs)
        d = (x8 - cx) ** 2 + (y8 - cy) ** 2 + (z8 - cz) ** 2
        dists = jnp.minimum(dists, d)
        far = jnp.argmax(dists).astype(jnp.int32)
        return dists, far, cxs, cys, czs, cidx

    init = (jnp.full((SUB, LAN), 1e10, jnp.float32), jnp.int32(0),
            jnp.zeros((G, 1), jnp.float32), jnp.zeros((G, 1), jnp.float32),
            jnp.zeros((G, 1), jnp.float32), jnp.zeros((1, G), jnp.int32))
    _, _, cxs, cys, czs, cidx = jax.lax.fori_loop(0, G, body, init)

    px = xyz_ref[0:1, :]
    py = xyz_ref[1:2, :]
    pz = xyz_ref[2:3, :]
    d2 = (cxs - px) ** 2 + (cys - py) ** 2 + (czs - pz) ** 2

    lane_iota_n = jax.lax.broadcasted_iota(jnp.int32, (G, N), 1)
    l_iota_gs = jax.lax.broadcasted_iota(jnp.int32, (G, GS), 1)
    ori = jnp.zeros((G, GS), jnp.int32)
    for k in range(GS):
        idxc = jnp.expand_dims(jnp.argmin(d2, axis=1).astype(jnp.int32), 1)
        ori = jnp.where(l_iota_gs == k, idxc, ori)
        d2 = jnp.where(lane_iota_n == idxc, BIGF, d2)

    center_ref[:, 0:1] = cxs
    center_ref[:, 1:2] = cys
    center_ref[:, 2:3] = czs
    ori_ref[:] = ori
    cidx_ref[:] = cidx


def _sc_gather_kernel(table_hbm, idx_hbm, out_hbm, idx_v, rows_v, sem):
    wid = jax.lax.axis_index("s") * NC_SC + jax.lax.axis_index("c")
    base = wid * BPW
    pltpu.sync_copy(idx_hbm.at[pl.ds(base, BPW)], idx_v)
    pltpu.async_copy(table_hbm.at[idx_v], rows_v, sem).wait()
    pltpu.sync_copy(rows_v, out_hbm.at[pl.ds(base, BPW)])


def _mlp_interp_kernel(xyz_ref, nb_ref, ctr_ref, w1_ref, b1_ref, w2_ref,
                       b2_ref, fgc_ref, out_ref, fgc_scr):
    @pl.when(pl.program_id(0) == 0)
    def _mlp():
        nb = nb_ref[:]
        cxa = jnp.concatenate([ctr_ref[:, 0:1]] * GS, axis=0)
        cya = jnp.concatenate([ctr_ref[:, 1:2]] * GS, axis=0)
        cza = jnp.concatenate([ctr_ref[:, 2:3]] * GS, axis=0)
        nx = nb[:, 0:1] - cxa
        ny = nb[:, 1:2] - cya
        nz = nb[:, 2:3] - cza
        w1 = w1_ref[:]
        h = jnp.maximum(
            nx * w1[0:1, :] + ny * w1[1:2, :] + nz * w1[2:3, :] + b1_ref[:],
            0.0)
        f = jnp.dot(h, w2_ref[:],
                    preferred_element_type=jnp.float32) + b2_ref[:]
        fgc = f[0:G, :]
        for k in range(1, GS):
            fgc = jnp.maximum(fgc, f[k * G:(k + 1) * G, :])
        fgc_ref[:] = fgc
        fgc_scr[:] = fgc

    px = xyz_ref[0:1, :]
    py = xyz_ref[1:2, :]
    pz = xyz_ref[2:3, :]
    cxs = ctr_ref[:, 0:1]
    cys = ctr_ref[:, 1:2]
    czs = ctr_ref[:, 2:3]
    d2 = (px - cxs) ** 2 + (py - cys) ** 2 + (pz - czs) ** 2
    g_iota = jax.lax.broadcasted_iota(jnp.int32, (G, NB), 0)
    rs = []
    gsel = []
    for _ in range(3):
        m = jnp.min(d2, axis=0, keepdims=True)
        gk = jnp.min(jnp.where(d2 == m, g_iota, G), axis=0, keepdims=True)
        dist = jnp.maximum(m, 1e-10)
        rs.append(1.0 / (dist + 1e-8))
        gsel.append(gk)
        d2 = jnp.where(g_iota == gk, BIGF, d2)
    rsum = (rs[0] + rs[1]) + rs[2]
    w = jnp.zeros((G, NB), jnp.float32)
    for k in range(3):
        w = w + jnp.where(g_iota == gsel[k], rs[k] / rsum, 0.0)
    out_ref[:] = jax.lax.dot_general(
        fgc_scr[:], w, (((0,), (0,)), ((), ())),
        preferred_element_type=jnp.float32)


def kernel(xyz, W1, b1, W2, b2):
    x = xyz[0]
    x8 = x[0].reshape(SUB, LAN)
    y8 = x[1].reshape(SUB, LAN)
    z8 = x[2].reshape(SUB, LAN)
    center, ori, cidx = pl.pallas_call(
        _fps_knn_kernel,
        out_shape=[
            jax.ShapeDtypeStruct((G, 3), jnp.float32),
            jax.ShapeDtypeStruct((G, GS), jnp.int32),
            jax.ShapeDtypeStruct((1, G), jnp.int32),
        ],
    )(x, x8, y8, z8)

    table = jnp.concatenate(
        [x.T, jnp.zeros((N, DPAD - 3), jnp.float32)], axis=1)
    idx_flat = ori.T.reshape(GS * G)
    nb = pl.kernel(
        _sc_gather_kernel,
        out_type=jax.ShapeDtypeStruct((GS * G, DPAD), jnp.float32),
        mesh=plsc.VectorSubcoreMesh(core_axis_name="c", subcore_axis_name="s"),
        scratch_types=[
            pltpu.VMEM((BPW,), jnp.int32),
            pltpu.VMEM((BPW, DPAD), jnp.float32),
            pltpu.SemaphoreType.DMA,
        ],
    )(table, idx_flat)

    fgc, interp = pl.pallas_call(
        _mlp_interp_kernel,
        grid=(N // NB,),
        in_specs=[
            pl.BlockSpec((3, NB), lambda i: (0, i)),
            pl.BlockSpec((GS * G, DPAD), lambda i: (0, 0)),
            pl.BlockSpec((G, 3), lambda i: (0, 0)),
            pl.BlockSpec((3, HID), lambda i: (0, 0)),
            pl.BlockSpec((1, HID), lambda i: (0, 0)),
            pl.BlockSpec((HID, C), lambda i: (0, 0)),
            pl.BlockSpec((1, C), lambda i: (0, 0)),
        ],
        out_specs=[
            pl.BlockSpec((G, C), lambda i: (0, 0)),
            pl.BlockSpec((C, NB), lambda i: (0, i)),
        ],
        out_shape=[
            jax.ShapeDtypeStruct((G, C), jnp.float32),
            jax.ShapeDtypeStruct((C, N), jnp.float32),
        ],
        scratch_shapes=[pltpu.VMEM((G, C), jnp.float32)],
    )(x, nb, center, W1, b1.reshape(1, HID), W2, b2.reshape(1, C))
    return (fgc.T[None], center[None], ori[None], cidx, interp[None])


# speedup vs baseline: 1.1280x; 1.0044x over previous
import jax
import jax.numpy as jnp
from jax.experimental import pallas as pl
from jax.experimental.pallas import tpu as pltpu
from jax.experimental.pallas import tpu_sc as plsc

GS = 32
G = 96
HID = 128
C = 384
N = 16384
SUB = 8
LAN = N // SUB
NB = 4096
BIGF = 3.4e38
DPAD = 128
NC_SC = 2
NS_SC = 16
NW = NC_SC * NS_SC
BPW = GS * G // NW


def _fps_knn_kernel(xyz_ref, x8_ref, y8_ref, z8_ref, center_ref, ori_ref,
                    cidx_ref):
    x8 = x8_ref[:]
    y8 = y8_ref[:]
    z8 = z8_ref[:]
    n_iota8 = (jax.lax.broadcasted_iota(jnp.int32, (SUB, LAN), 0) * LAN
               + jax.lax.broadcasted_iota(jnp.int32, (SUB, LAN), 1))
    s_iota_col = jax.lax.broadcasted_iota(jnp.int32, (G, 1), 0)
    row_iota_g = jax.lax.broadcasted_iota(jnp.int32, (1, G), 1)

    def body(i, carry):
        dists, far, cxs, cys, czs, cidx = carry
        cidx = jnp.where(row_iota_g == i, far, cidx)
        sel = n_iota8 == far
        cx = jnp.sum(jnp.where(sel, x8, 0.0))
        cy = jnp.sum(jnp.where(sel, y8, 0.0))
        cz = jnp.sum(jnp.where(sel, z8, 0.0))
        cxs = jnp.where(s_iota_col == i, cx, cxs)
        cys = jnp.where(s_iota_col == i, cy, cys)
        czs = jnp.where(s_iota_col == i, cz, czs)
        d = (x8 - cx) ** 2 + (y8 - cy) ** 2 + (z8 - cz) ** 2
        dists = jnp.minimum(dists, d)
        far = jnp.argmax(dists).astype(jnp.int32)
        return dists, far, cxs, cys, czs, cidx

    init = (jnp.full((SUB, LAN), 1e10, jnp.float32), jnp.int32(0),
            jnp.zeros((G, 1), jnp.float32), jnp.zeros((G, 1), jnp.float32),
            jnp.zeros((G, 1), jnp.float32), jnp.zeros((1, G), jnp.int32))
    _, _, cxs, cys, czs, cidx = jax.lax.fori_loop(0, G, body, init)

    px = xyz_ref[0:1, :]
    py = xyz_ref[1:2, :]
    pz = xyz_ref[2:3, :]
    d2 = (cxs - px) ** 2 + (cys - py) ** 2 + (czs - pz) ** 2

    lane_iota_n = jax.lax.broadcasted_iota(jnp.int32, (G, N), 1)
    l_iota_gs = jax.lax.broadcasted_iota(jnp.int32, (G, GS), 1)
    ori = jnp.zeros((G, GS), jnp.int32)
    for k in range(GS):
        idxc = jnp.expand_dims(jnp.argmin(d2, axis=1).astype(jnp.int32), 1)
        ori = jnp.where(l_iota_gs == k, idxc, ori)
        d2 = jnp.where(lane_iota_n == idxc, BIGF, d2)

    center_ref[:, 0:1] = cxs
    center_ref[:, 1:2] = cys
    center_ref[:, 2:3] = czs
    ori_ref[:] = ori
    cidx_ref[:] = cidx


def _sc_gather_kernel(table_hbm, idx_hbm, out_hbm, idx_v, rows_v, sem):
    wid = jax.lax.axis_index("s") * NC_SC + jax.lax.axis_index("c")
    base = wid * BPW
    pltpu.sync_copy(idx_hbm.at[pl.ds(base, BPW)], idx_v)
    pltpu.async_copy(table_hbm.at[idx_v], rows_v, sem).wait()
    pltpu.sync_copy(rows_v, out_hbm.at[pl.ds(base, BPW)])


def _mlp_interp_kernel(xyz_ref, nb_ref, ctr_ref, w1_ref, b1_ref, w2_ref,
                       b2_ref, fgc_ref, out_ref, fgc_scr):
    @pl.when(pl.program_id(0) == 0)
    def _mlp():
        nb = nb_ref[:]
        cxa = jnp.concatenate([ctr_ref[:, 0:1]] * GS, axis=0)
        cya = jnp.concatenate([ctr_ref[:, 1:2]] * GS, axis=0)
        cza = jnp.concatenate([ctr_ref[:, 2:3]] * GS, axis=0)
        nx = nb[:, 0:1] - cxa
        ny = nb[:, 1:2] - cya
        nz = nb[:, 2:3] - cza
        w1 = w1_ref[:]
        h = jnp.maximum(
            nx * w1[0:1, :] + ny * w1[1:2, :] + nz * w1[2:3, :] + b1_ref[:],
            0.0)
        f = jnp.dot(h, w2_ref[:],
                    preferred_element_type=jnp.float32) + b2_ref[:]
        fgc = f[0:G, :]
        for k in range(1, GS):
            fgc = jnp.maximum(fgc, f[k * G:(k + 1) * G, :])
        fgc_ref[:] = fgc
        fgc_scr[:] = fgc

    px = xyz_ref[0:1, :]
    py = xyz_ref[1:2, :]
    pz = xyz_ref[2:3, :]
    cxs = ctr_ref[:, 0:1]
    cys = ctr_ref[:, 1:2]
    czs = ctr_ref[:, 2:3]
    d2 = (px - cxs) ** 2 + (py - cys) ** 2 + (pz - czs) ** 2
    g_iota = jax.lax.broadcasted_iota(jnp.int32, (G, NB), 0)
    rs = []
    gsel = []
    for _ in range(3):
        m = jnp.min(d2, axis=0, keepdims=True)
        gk = jnp.min(jnp.where(d2 == m, g_iota, G), axis=0, keepdims=True)
        dist = jnp.maximum(m, 1e-10)
        rs.append(1.0 / (dist + 1e-8))
        gsel.append(gk)
        d2 = jnp.where(g_iota == gk, BIGF, d2)
    rsum = (rs[0] + rs[1]) + rs[2]
    w = jnp.zeros((G, NB), jnp.float32)
    for k in range(3):
        w = w + jnp.where(g_iota == gsel[k], rs[k] / rsum, 0.0)
    out_ref[:] = jax.lax.dot_general(
        fgc_scr[:], w, (((0,), (0,)), ((), ())),
        preferred_element_type=jnp.float32)


def kernel(xyz, W1, b1, W2, b2):
    x = xyz[0]
    x8 = x[0].reshape(SUB, LAN)
    y8 = x[1].reshape(SUB, LAN)
    z8 = x[2].reshape(SUB, LAN)
    center, ori, cidx = pl.pallas_call(
        _fps_knn_kernel,
        out_shape=[
            jax.ShapeDtypeStruct((G, 3), jnp.float32),
            jax.ShapeDtypeStruct((G, GS), jnp.int32),
            jax.ShapeDtypeStruct((1, G), jnp.int32),
        ],
    )(x, x8, y8, z8)

    table = jnp.concatenate(
        [x.T, jnp.zeros((N, DPAD - 3), jnp.float32)], axis=1)
    idx_flat = ori.T.reshape(GS * G)
    nb = pl.kernel(
        _sc_gather_kernel,
        out_type=jax.ShapeDtypeStruct((GS * G, DPAD), jnp.float32),
        mesh=plsc.VectorSubcoreMesh(core_axis_name="c", subcore_axis_name="s"),
        scratch_types=[
            pltpu.VMEM((BPW,), jnp.int32),
            pltpu.VMEM((BPW, DPAD), jnp.float32),
            pltpu.SemaphoreType.DMA,
        ],
    )(table, idx_flat)

    fgc, interp = pl.pallas_call(
        _mlp_interp_kernel,
        grid=(N // NB,),
        in_specs=[
            pl.BlockSpec((3, NB), lambda i: (0, i)),
            pl.BlockSpec((GS * G, DPAD), lambda i: (0, 0)),
            pl.BlockSpec((G, 3), lambda i: (0, 0)),
            pl.BlockSpec((3, HID), lambda i: (0, 0)),
            pl.BlockSpec((1, HID), lambda i: (0, 0)),
            pl.BlockSpec((HID, C), lambda i: (0, 0)),
            pl.BlockSpec((1, C), lambda i: (0, 0)),
        ],
        out_specs=[
            pl.BlockSpec((G, C), lambda i: (0, 0)),
            pl.BlockSpec((C, NB), lambda i: (0, i)),
        ],
        out_shape=[
            jax.ShapeDtypeStruct((G, C), jnp.float32),
            jax.ShapeDtypeStruct((C, N), jnp.float32),
        ],
        scratch_shapes=[pltpu.VMEM((G, C), jnp.float32)],
    )(x, nb, center, W1, b1.reshape(1, HID), W2, b2.reshape(1, C))
    return (fgc.T[None], center[None], ori[None], cidx, interp[None])
